# baseline (speedup 1.0000x reference)
import functools

import jax
import jax.numpy as jnp
from jax import lax
from jax.experimental import pallas as pl
from jax.experimental.pallas import tpu as pltpu
from jax.experimental.pallas import tpu_sc as plsc

_SC_PARAMS = pltpu.CompilerParams(needs_layout_passes=False,
                                  use_tc_tiling_on_sc=False)

N_CON = 5000
N_COL = 5000
N = N_CON + N_COL
E = 160000
HIDDEN = 128
HEADS = 8
EMBED = 64

NC = 2
NS = 16

D1 = HEADS * HIDDEN
R1 = 1296
P1 = 4
R1_PAD = R1 + 16

D2 = HIDDEN
R2 = 2512
R2_PAD = R2 + 16

EPT = E // NS
EPW = E // (NC * NS)
CH = 16
SB = 2000


def _i16():
    return jnp.arange(16, dtype=jnp.int32)


def _leaky(x):
    return jnp.where(x > 0, x, 0.2 * x)


def _splat(v, i):
    idx = jnp.full((16,), i, jnp.int32)
    return v.at[idx].get(mode="promise_in_bounds")


def _tc1_body(cs_ref, cols_ref, wn_ref, bn_ref, wc_ref, bc_ref, w1_ref,
              a32_ref, xp_ref, att_ref):
    i = pl.program_id(0)
    half = pl.num_programs(0) // 2

    def compute(x, w, b):
        emb = jax.nn.relu(
            jnp.dot(x, w, preferred_element_type=jnp.float32) + b)
        xp_ref[...] = jnp.dot(emb, w1_ref[...],
                              preferred_element_type=jnp.float32)
        att_ref[...] = jnp.dot(emb, a32_ref[...],
                               preferred_element_type=jnp.float32)

    @pl.when(i < half)
    def _():
        compute(cs_ref[...], wn_ref[...], bn_ref[...])

    @pl.when(i >= half)
    def _():
        compute(cols_ref[...], wc_ref[...], bc_ref[...])


def _tc1(cs, cols, Wnf, bn, Wcf, bc, W1, A32):
    blk = 1000
    nblk = N // blk
    half = nblk // 2
    return pl.pallas_call(
        _tc1_body,
        grid=(nblk,),
        in_specs=[
            pl.BlockSpec((blk, HIDDEN), lambda i: (jnp.minimum(i, half - 1), 0)),
            pl.BlockSpec((blk, HIDDEN), lambda i: (jnp.maximum(i - half, 0), 0)),
            pl.BlockSpec((HIDDEN, HIDDEN), lambda i: (0, 0)),
            pl.BlockSpec((HIDDEN,), lambda i: (0,)),
            pl.BlockSpec((HIDDEN, HIDDEN), lambda i: (0, 0)),
            pl.BlockSpec((HIDDEN,), lambda i: (0,)),
            pl.BlockSpec((HIDDEN, D1), lambda i: (0, 0)),
            pl.BlockSpec((HIDDEN, 32), lambda i: (0, 0)),
        ],
        out_specs=[
            pl.BlockSpec((blk, D1), lambda i: (i, 0)),
            pl.BlockSpec((blk, 32), lambda i: (i, 0)),
        ],
        out_shape=[
            jax.ShapeDtypeStruct((N, D1), jnp.float32),
            jax.ShapeDtypeStruct((N, 32), jnp.float32),
        ],
    )(cs, cols, Wnf, bn, Wcf, bc, W1, A32)


def _a1_body(asrc_hbm, adst_hbm, src_hbm, dst_hbm, w_hbm,
             src_t, dst_t, abuf, bbuf, wout, sem_a, sem_b):
    c = lax.axis_index("c")
    s = lax.axis_index("s")
    wid = s * NC + c
    base = wid * EPW
    pltpu.sync_copy(src_hbm.at[pl.ds(base, EPW)], src_t)
    pltpu.sync_copy(dst_hbm.at[pl.ds(base, EPW)], dst_t)
    nchunk = EPW // 128

    def chunk(k, carry):
        ca = pltpu.make_async_copy(
            asrc_hbm.at[src_t.at[pl.ds(k * 128, 128)]], abuf, sem_a)
        ca.start()
        cb = pltpu.make_async_copy(
            adst_hbm.at[dst_t.at[pl.ds(k * 128, 128)]], bbuf, sem_b)
        cb.start()
        ca.wait()
        cb.wait()
        for r in range(128):
            wout[pl.ds(r * 16, 16)] = jnp.exp(_leaky(abuf[r] + bbuf[r]))
        pltpu.sync_copy(wout.at[pl.ds(0, 2048)],
                        w_hbm.at[pl.ds((base + k * 128) * 16, 2048)])
        return carry

    lax.fori_loop(0, nchunk, chunk, 0)

    tail = EPW - nchunk * 128
    if tail:
        toff = nchunk * 128
        ca = pltpu.make_async_copy(
            asrc_hbm.at[src_t.at[pl.ds(toff, tail)]],
            abuf.at[pl.ds(0, tail)], sem_a)
        ca.start()
        cb = pltpu.make_async_copy(
            adst_hbm.at[dst_t.at[pl.ds(toff, tail)]],
            bbuf.at[pl.ds(0, tail)], sem_b)
        cb.start()
        ca.wait()
        cb.wait()
        for r in range(tail):
            wout[pl.ds(r * 16, 16)] = jnp.exp(_leaky(abuf[r] + bbuf[r]))
        pltpu.sync_copy(wout.at[pl.ds(0, tail * 16)],
                        w_hbm.at[pl.ds((base + toff) * 16, tail * 16)])


def _a1(asrc16, adst16, src, dst):
    mesh = plsc.VectorSubcoreMesh(core_axis_name="c", subcore_axis_name="s")
    f = pl.kernel(
        _a1_body,
        out_type=jax.ShapeDtypeStruct((E * 16,), jnp.float32),
        mesh=mesh,
        scratch_types=[
            pltpu.VMEM((EPW,), jnp.int32),
            pltpu.VMEM((EPW,), jnp.int32),
            pltpu.VMEM((128, 16), jnp.float32),
            pltpu.VMEM((128, 16), jnp.float32),
            pltpu.VMEM((2048,), jnp.float32),
            pltpu.SemaphoreType.DMA,
            pltpu.SemaphoreType.DMA,
        ],
        compiler_params=_SC_PARAMS,
    )
    return f(asrc16, adst16, src, dst)


def _b1_body(xp_hbm, w_hbm, src_hbm, dst_hbm, acc_hbm, den_hbm,
             srcb, dstb, loc_l, src_l, eid_l,
             xbA, xbB, wbA, wbB, zbuf, acc_sh, den_sh,
             semGA, semGB, semSA, semSB):
    dv = D1 // 16
    zrows = R1_PAD // NS
    frows = R1 // NS
    c = lax.axis_index("c")
    s = lax.axis_index("s")

    def zzb(i, carry):
        zbuf[i, pl.ds(0, 16)] = jnp.zeros((16,), jnp.float32)
        return carry
    lax.fori_loop(0, zrows, zzb, 0)

    def gstart(xb, wb, srcv, eidv, sem):
        pltpu.async_copy(xp_hbm.at[srcv], xb, sem)
        pltpu.async_copy(w_hbm.at[eidv], wb, sem)

    def gwait(xb, wb, sem):
        pltpu.make_async_copy(xp_hbm.at[pl.ds(0, CH)], xb, sem).wait()
        pltpu.make_async_copy(w_hbm.at[pl.ds(0, CH)], wb, sem).wait()

    def compute(xb, wb):
        def edge(e, carry):
            wv = wb[e]
            for h in range(HEADS):
                ws = _splat(wv, h)
                for v in range(HIDDEN // 16):
                    col = h * HIDDEN + v * 16
                    xb[e, pl.ds(col, 16)] = xb[e, pl.ds(col, 16)] * ws
            return carry
        lax.fori_loop(0, CH, edge, 0)

    def sstart(xb, wb, locv, sem):
        pltpu.async_copy(xb, acc_sh.at[locv], sem, add=True)
        pltpu.async_copy(wb, den_sh.at[locv], sem, add=True)

    def sdrain(xb, wb, sem):
        pltpu.make_async_copy(xb, acc_sh.at[pl.ds(0, CH)], sem).wait()
        pltpu.make_async_copy(wb, den_sh.at[pl.ds(0, CH)], sem).wait()

    def lists(i):
        return (src_l[pl.ds(i * CH, CH)], eid_l[pl.ds(i * CH, CH)],
                loc_l[pl.ds(i * CH, CH)])

    def one_pass(p, pcarry):
        lo = (p * NC + c) * R1

        def zx(v, carry):
            zero16 = jnp.zeros((16,), jnp.float32)
            for r in range(CH):
                xbA[r, pl.ds(v * 16, 16)] = zero16
            return carry
        lax.fori_loop(0, dv, zx, 0)
        nzc = zrows // CH
        ztail = zrows - nzc * CH
        for z in range(nzc):
            pltpu.sync_copy(xbA, acc_sh.at[pl.ds(s * zrows + z * CH, CH)])
        if ztail:
            pltpu.sync_copy(xbA.at[pl.ds(0, ztail)],
                            acc_sh.at[pl.ds(s * zrows + nzc * CH, ztail)])
        pltpu.sync_copy(zbuf, den_sh.at[pl.ds(s * zrows, zrows)])
        plsc.subcore_barrier()

        def one_block(q, qcarry):
            base = s * EPT + q * SB
            pltpu.sync_copy(src_hbm.at[pl.ds(base, SB)], srcb)
            pltpu.sync_copy(dst_hbm.at[pl.ds(base, SB)], dstb)

            def scan(i, cnt):
                d = dstb[pl.ds(i * 16, 16)]
                m = (d >= lo) & (d < lo + R1)
                plsc.store_compressed(loc_l.at[pl.ds(cnt, 16)], d - lo,
                                      mask=m)
                plsc.store_compressed(src_l.at[pl.ds(cnt, 16)],
                                      srcb[pl.ds(i * 16, 16)], mask=m)
                plsc.store_compressed(eid_l.at[pl.ds(cnt, 16)],
                                      base + i * 16 + _i16(), mask=m)
                return cnt + jnp.sum(m.astype(jnp.int32))
            cnt = lax.fori_loop(0, SB // 16, scan, 0)

            for t in range(2):
                loc_l[pl.ds(cnt + t * 16, 16)] = jnp.full(
                    (16,), R1, jnp.int32)
                src_l[pl.ds(cnt + t * 16, 16)] = jnp.zeros((16,), jnp.int32)
                eid_l[pl.ds(cnt + t * 16, 16)] = jnp.zeros((16,), jnp.int32)
            npair = (cnt + 2 * CH - 1) // (2 * CH)

            @pl.when(npair > 0)
            def _():
                sa, ea, la = lists(0)
                gstart(xbA, wbA, sa, ea, semGA)
                sb_, eb, lb = lists(1)
                gstart(xbB, wbB, sb_, eb, semGB)

            def pair(j, carry):
                sa, ea, la = lists(2 * j)
                gwait(xbA, wbA, semGA)
                compute(xbA, wbA)
                sstart(xbA, wbA, la, semSA)
                sb_, eb, lb = lists(2 * j + 1)
                gwait(xbB, wbB, semGB)
                compute(xbB, wbB)
                sstart(xbB, wbB, lb, semSB)

                @pl.when(j < npair - 1)
                def _():
                    sn, en, ln = lists(2 * j + 2)
                    sdrain(xbA, wbA, semSA)
                    gstart(xbA, wbA, sn, en, semGA)
                    sn2, en2, ln2 = lists(2 * j + 3)
                    sdrain(xbB, wbB, semSB)
                    gstart(xbB, wbB, sn2, en2, semGB)
                return carry
            lax.fori_loop(0, npair, pair, 0)

            @pl.when(npair > 0)
            def _():
                sdrain(xbA, wbA, semSA)
                sdrain(xbB, wbB, semSB)
            return qcarry
        lax.fori_loop(0, EPT // SB, one_block, 0)
        plsc.subcore_barrier()

        pltpu.sync_copy(acc_sh.at[pl.ds(s * frows, frows)],
                        acc_hbm.at[pl.ds(lo + s * frows, frows)])
        pltpu.sync_copy(den_sh.at[pl.ds(s * frows, frows)],
                        den_hbm.at[pl.ds(lo + s * frows, frows)])
        plsc.subcore_barrier()
        return pcarry
    lax.fori_loop(0, P1, one_pass, 0)


def _run_b1(xp, w, src, dst):
    mesh = plsc.VectorSubcoreMesh(core_axis_name="c", subcore_axis_name="s")
    out_rows = 2 * P1 * R1
    f = pl.kernel(
        _b1_body,
        out_type=[
            jax.ShapeDtypeStruct((out_rows, D1), jnp.float32),
            jax.ShapeDtypeStruct((out_rows, 16), jnp.float32),
        ],
        mesh=mesh,
        scratch_types=[
            pltpu.VMEM((SB,), jnp.int32),
            pltpu.VMEM((SB,), jnp.int32),
            pltpu.VMEM((SB + 64,), jnp.int32),
            pltpu.VMEM((SB + 64,), jnp.int32),
            pltpu.VMEM((SB + 64,), jnp.int32),
            pltpu.VMEM((CH, D1), jnp.float32),
            pltpu.VMEM((CH, D1), jnp.float32),
            pltpu.VMEM((CH, 16), jnp.float32),
            pltpu.VMEM((CH, 16), jnp.float32),
            pltpu.VMEM((R1_PAD // NS, 16), jnp.float32),
            pltpu.VMEM_SHARED((R1_PAD, D1), jnp.float32),
            pltpu.VMEM_SHARED((R1_PAD, 16), jnp.float32),
            pltpu.SemaphoreType.DMA,
            pltpu.SemaphoreType.DMA,
            pltpu.SemaphoreType.DMA,
            pltpu.SemaphoreType.DMA,
        ],
        compiler_params=_SC_PARAMS,
    )
    return f(xp, w, src, dst)


def _b2_body(xp_hbm, asrc_hbm, adst_hbm, src_hbm, dst_hbm, acc_hbm, den_hbm,
             srcb, dstb, loc_l, src_l,
             xbA, xbB, wbA, wbB, zbuf, asrc_t, adst_t, acc_sh, den_sh,
             semGA, semGB, semSA, semSB):
    dv = D2 // 16
    zrows = R2_PAD // NS
    frows = R2 // NS
    c = lax.axis_index("c")
    s = lax.axis_index("s")
    lo = c * R2 + N_CON
    pltpu.sync_copy(asrc_hbm, asrc_t)
    pltpu.sync_copy(adst_hbm, adst_t)

    def zzb(i, carry):
        zbuf[i, pl.ds(0, 16)] = jnp.zeros((16,), jnp.float32)
        return carry
    lax.fori_loop(0, zrows, zzb, 0)

    def zx(v, carry):
        zero16 = jnp.zeros((16,), jnp.float32)
        for r in range(CH):
            xbA[r, pl.ds(v * 16, 16)] = zero16
        return carry
    lax.fori_loop(0, dv, zx, 0)
    nzc = zrows // CH
    ztail = zrows - nzc * CH
    for z in range(nzc):
        pltpu.sync_copy(xbA, acc_sh.at[pl.ds(s * zrows + z * CH, CH)])
    if ztail:
        pltpu.sync_copy(xbA.at[pl.ds(0, ztail)],
                        acc_sh.at[pl.ds(s * zrows + nzc * CH, ztail)])
    pltpu.sync_copy(zbuf, den_sh.at[pl.ds(s * zrows, zrows)])
    plsc.subcore_barrier()

    def gstart(xb, wb, srcv, locv, sem):
        pltpu.async_copy(xp_hbm.at[srcv], xb, sem)
        av = plsc.load_gather(asrc_t, [srcv])
        bv = plsc.load_gather(adst_t, [jnp.minimum(locv + lo, N - 1)])
        w16 = jnp.exp(_leaky(av + bv))

        def we(e, carry):
            wb[e, pl.ds(0, 16)] = _splat(w16, e)
            return carry
        lax.fori_loop(0, CH, we, 0)

    def gwait(xb, sem):
        pltpu.make_async_copy(xp_hbm.at[pl.ds(0, CH)], xb, sem).wait()

    def compute(xb, wb):
        def edge(e, carry):
            ws = wb[e]
            for v in range(dv):
                xb[e, pl.ds(v * 16, 16)] = xb[e, pl.ds(v * 16, 16)] * ws
            return carry
        lax.fori_loop(0, CH, edge, 0)

    def sstart(xb, wb, locv, sem):
        pltpu.async_copy(xb, acc_sh.at[locv], sem, add=True)
        pltpu.async_copy(wb, den_sh.at[locv], sem, add=True)

    def sdrain(xb, wb, sem):
        pltpu.make_async_copy(xb, acc_sh.at[pl.ds(0, CH)], sem).wait()
        pltpu.make_async_copy(wb, den_sh.at[pl.ds(0, CH)], sem).wait()

    def lists(i):
        return (src_l[pl.ds(i * CH, CH)], loc_l[pl.ds(i * CH, CH)])

    def one_block(q, qcarry):
        base = s * EPT + q * SB
        pltpu.sync_copy(src_hbm.at[pl.ds(base, SB)], srcb)
        pltpu.sync_copy(dst_hbm.at[pl.ds(base, SB)], dstb)

        def scan(i, cnt):
            d = dstb[pl.ds(i * 16, 16)]
            m = (d >= lo) & (d < lo + R2)
            plsc.store_compressed(loc_l.at[pl.ds(cnt, 16)], d - lo, mask=m)
            plsc.store_compressed(src_l.at[pl.ds(cnt, 16)],
                                  srcb[pl.ds(i * 16, 16)], mask=m)
            return cnt + jnp.sum(m.astype(jnp.int32))
        cnt = lax.fori_loop(0, SB // 16, scan, 0)

        for t in range(2):
            loc_l[pl.ds(cnt + t * 16, 16)] = jnp.full((16,), R2, jnp.int32)
            src_l[pl.ds(cnt + t * 16, 16)] = jnp.zeros((16,), jnp.int32)
        npair = (cnt + 2 * CH - 1) // (2 * CH)

        @pl.when(npair > 0)
        def _():
            sa, la = lists(0)
            gstart(xbA, wbA, sa, la, semGA)
            sb_, lb = lists(1)
            gstart(xbB, wbB, sb_, lb, semGB)

        def pair(j, carry):
            sa, la = lists(2 * j)
            gwait(xbA, semGA)
            compute(xbA, wbA)
            sstart(xbA, wbA, la, semSA)
            sb_, lb = lists(2 * j + 1)
            gwait(xbB, semGB)
            compute(xbB, wbB)
            sstart(xbB, wbB, lb, semSB)

            @pl.when(j < npair - 1)
            def _():
                sn, ln = lists(2 * j + 2)
                sdrain(xbA, wbA, semSA)
                gstart(xbA, wbA, sn, ln, semGA)
                sn2, ln2 = lists(2 * j + 3)
                sdrain(xbB, wbB, semSB)
                gstart(xbB, wbB, sn2, ln2, semGB)
            return carry
        lax.fori_loop(0, npair, pair, 0)

        @pl.when(npair > 0)
        def _():
            sdrain(xbA, wbA, semSA)
            sdrain(xbB, wbB, semSB)
        return qcarry
    lax.fori_loop(0, EPT // SB, one_block, 0)
    plsc.subcore_barrier()

    pltpu.sync_copy(acc_sh.at[pl.ds(s * frows, frows)],
                    acc_hbm.at[pl.ds(c * R2 + s * frows, frows)])
    pltpu.sync_copy(den_sh.at[pl.ds(s * frows, frows)],
                    den_hbm.at[pl.ds(c * R2 + s * frows, frows)])
    plsc.subcore_barrier()


def _run_b2(xp2, asrc2, adst2, src, dst):
    mesh = plsc.VectorSubcoreMesh(core_axis_name="c", subcore_axis_name="s")
    f = pl.kernel(
        _b2_body,
        out_type=[
            jax.ShapeDtypeStruct((2 * R2, D2), jnp.float32),
            jax.ShapeDtypeStruct((2 * R2, 16), jnp.float32),
        ],
        mesh=mesh,
        scratch_types=[
            pltpu.VMEM((SB,), jnp.int32),
            pltpu.VMEM((SB,), jnp.int32),
            pltpu.VMEM((SB + 64,), jnp.int32),
            pltpu.VMEM((SB + 64,), jnp.int32),
            pltpu.VMEM((CH, D2), jnp.float32),
            pltpu.VMEM((CH, D2), jnp.float32),
            pltpu.VMEM((CH, 16), jnp.float32),
            pltpu.VMEM((CH, 16), jnp.float32),
            pltpu.VMEM((R2_PAD // NS, 16), jnp.float32),
            pltpu.VMEM((N,), jnp.float32),
            pltpu.VMEM((N,), jnp.float32),
            pltpu.VMEM_SHARED((R2_PAD, D2), jnp.float32),
            pltpu.VMEM_SHARED((R2_PAD, 16), jnp.float32),
            pltpu.SemaphoreType.DMA,
            pltpu.SemaphoreType.DMA,
            pltpu.SemaphoreType.DMA,
            pltpu.SemaphoreType.DMA,
        ],
        compiler_params=_SC_PARAMS,
    )
    return f(xp2, asrc2, adst2, src, dst)


def _tc2_body(acc_ref, den_ref, b1_ref, w2_ref, a2_ref, xp2_ref, att2_ref):
    feat = acc_ref[...].reshape(-1, HEADS, HIDDEN)
    den = den_ref[...][:, :HEADS].reshape(-1, HEADS, 1)
    emb = jax.nn.relu((feat / (den + 1e-16)).reshape(-1, D1) + b1_ref[...])
    xp2_ref[...] = jnp.dot(emb, w2_ref[...],
                           preferred_element_type=jnp.float32)
    att2_ref[...] = jnp.dot(emb, a2_ref[...],
                            preferred_element_type=jnp.float32)


def _tc2(acc1, den1, b1, W2, A2):
    blk = 1000
    return pl.pallas_call(
        _tc2_body,
        grid=(N // blk,),
        in_specs=[
            pl.BlockSpec((blk, D1), lambda i: (i, 0)),
            pl.BlockSpec((blk, 16), lambda i: (i, 0)),
            pl.BlockSpec((D1,), lambda i: (0,)),
            pl.BlockSpec((D1, HIDDEN), lambda i: (0, 0)),
            pl.BlockSpec((D1, 8), lambda i: (0, 0)),
        ],
        out_specs=[
            pl.BlockSpec((blk, HIDDEN), lambda i: (i, 0)),
            pl.BlockSpec((blk, 8), lambda i: (i, 0)),
        ],
        out_shape=[
            jax.ShapeDtypeStruct((N, HIDDEN), jnp.float32),
            jax.ShapeDtypeStruct((N, 8), jnp.float32),
        ],
    )(acc1, den1, b1, W2, A2)


def _tc3_body(acc_ref, den_ref, b2_ref, wo_ref, bo_ref, out_ref):
    den = den_ref[...][:, 0:1]
    emb = jax.nn.relu(acc_ref[...] / (den + 1e-16) + b2_ref[...])
    out_ref[...] = jnp.dot(emb, wo_ref[...],
                           preferred_element_type=jnp.float32) + bo_ref[...]


def _tc3(acc2, den2, b2, Wo, bo):
    blk = 1000
    return pl.pallas_call(
        _tc3_body,
        grid=(N_COL // blk,),
        in_specs=[
            pl.BlockSpec((blk, D2), lambda i: (i, 0)),
            pl.BlockSpec((blk, 16), lambda i: (i, 0)),
            pl.BlockSpec((HIDDEN,), lambda i: (0,)),
            pl.BlockSpec((HIDDEN, EMBED), lambda i: (0, 0)),
            pl.BlockSpec((EMBED,), lambda i: (0,)),
        ],
        out_specs=pl.BlockSpec((blk, EMBED), lambda i: (i, 0)),
        out_shape=jax.ShapeDtypeStruct((N_COL, EMBED), jnp.float32),
    )(acc2, den2, b2, Wo, bo)


def kernel(constraints_state, columns_state, edges, Wn, bn, Wc, bc, W1,
           a_src1, a_dst1, b1, W2, a_src2, a_dst2, b2, Wo, bo):
    Wnf = Wn[:HIDDEN] + Wn[HIDDEN:]
    Wcf = Wc[:HIDDEN] + Wc[HIDDEN:]
    As1 = jnp.einsum("khd,hd->kh", W1.reshape(HIDDEN, HEADS, HIDDEN), a_src1)
    Ad1 = jnp.einsum("khd,hd->kh", W1.reshape(HIDDEN, HEADS, HIDDEN), a_dst1)
    A32 = jnp.concatenate([As1, As1, Ad1, Ad1], axis=1)
    As2 = W2 @ a_src2[0]
    Ad2 = W2 @ a_dst2[0]
    A2 = jnp.stack([As2, Ad2] + [jnp.zeros_like(As2)] * 6, axis=1)

    src = edges[0]
    dst = edges[1]

    xp1, att = _tc1(constraints_state, columns_state, Wnf, bn, Wcf, bc,
                    W1, A32)
    w1flat = _a1(att[:, :16], att[:, 16:], src, dst)
    w1 = w1flat.reshape(E, 16)
    acc1, den1 = _run_b1(xp1, w1, src, dst)
    xp2, att2 = _tc2(acc1, den1, b1, W2, A2)
    acc2, den2 = _run_b2(xp2, att2[:, 0], att2[:, 1], src, dst)
    return _tc3(acc2, den2, b2, Wo, bo)


# speedup vs baseline: 1.5710x; 1.5710x over previous
import functools

import jax
import jax.numpy as jnp
from jax import lax
from jax.experimental import pallas as pl
from jax.experimental.pallas import tpu as pltpu
from jax.experimental.pallas import tpu_sc as plsc

_SC_PARAMS = pltpu.CompilerParams(needs_layout_passes=False,
                                  use_tc_tiling_on_sc=False)

N_CON = 5000
N_COL = 5000
N = N_CON + N_COL
E = 160000
HIDDEN = 128
HEADS = 8
EMBED = 64

NC = 2
NS = 16

D1 = HEADS * HIDDEN
R1 = 1296
P1 = 4
R1_PAD = R1 + 16

D2 = HIDDEN
R2 = 2512
R2_PAD = R2 + 16

EPT = E // NS
EPW = E // (NC * NS)
CH = 16
SB = 2000


def _i16():
    return jnp.arange(16, dtype=jnp.int32)


def _leaky(x):
    return jnp.where(x > 0, x, 0.2 * x)


def _splat(v, i):
    idx = jnp.full((16,), i, jnp.int32)
    return v.at[idx].get(mode="promise_in_bounds")


def _tc1_body(cs_ref, cols_ref, wn_ref, bn_ref, wc_ref, bc_ref, w1_ref,
              a32_ref, xp_ref, att_ref):
    i = pl.program_id(0)
    half = pl.num_programs(0) // 2

    def compute(x, w, b):
        emb = jax.nn.relu(
            jnp.dot(x, w, preferred_element_type=jnp.float32) + b)
        xp_ref[...] = jnp.dot(emb, w1_ref[...],
                              preferred_element_type=jnp.float32)
        att_ref[...] = jnp.dot(emb, a32_ref[...],
                               preferred_element_type=jnp.float32)

    @pl.when(i < half)
    def _():
        compute(cs_ref[...], wn_ref[...], bn_ref[...])

    @pl.when(i >= half)
    def _():
        compute(cols_ref[...], wc_ref[...], bc_ref[...])


def _tc1(cs, cols, Wnf, bn, Wcf, bc, W1, A32):
    blk = 1000
    nblk = N // blk
    half = nblk // 2
    return pl.pallas_call(
        _tc1_body,
        grid=(nblk,),
        in_specs=[
            pl.BlockSpec((blk, HIDDEN), lambda i: (jnp.minimum(i, half - 1), 0)),
            pl.BlockSpec((blk, HIDDEN), lambda i: (jnp.maximum(i - half, 0), 0)),
            pl.BlockSpec((HIDDEN, HIDDEN), lambda i: (0, 0)),
            pl.BlockSpec((HIDDEN,), lambda i: (0,)),
            pl.BlockSpec((HIDDEN, HIDDEN), lambda i: (0, 0)),
            pl.BlockSpec((HIDDEN,), lambda i: (0,)),
            pl.BlockSpec((HIDDEN, D1), lambda i: (0, 0)),
            pl.BlockSpec((HIDDEN, 32), lambda i: (0, 0)),
        ],
        out_specs=[
            pl.BlockSpec((blk, D1), lambda i: (i, 0)),
            pl.BlockSpec((blk, 32), lambda i: (i, 0)),
        ],
        out_shape=[
            jax.ShapeDtypeStruct((N, D1), jnp.float32),
            jax.ShapeDtypeStruct((N, 32), jnp.float32),
        ],
    )(cs, cols, Wnf, bn, Wcf, bc, W1, A32)


def _a1_body(asrc_hbm, adst_hbm, src_hbm, dst_hbm, w_hbm,
             src_t, dst_t, abuf, bbuf, wout, sem_a, sem_b):
    c = lax.axis_index("c")
    s = lax.axis_index("s")
    wid = s * NC + c
    base = wid * EPW
    pltpu.sync_copy(src_hbm.at[pl.ds(base, EPW)], src_t)
    pltpu.sync_copy(dst_hbm.at[pl.ds(base, EPW)], dst_t)
    nchunk = EPW // 128

    def chunk(k, carry):
        ca = pltpu.make_async_copy(
            asrc_hbm.at[src_t.at[pl.ds(k * 128, 128)]], abuf, sem_a)
        ca.start()
        cb = pltpu.make_async_copy(
            adst_hbm.at[dst_t.at[pl.ds(k * 128, 128)]], bbuf, sem_b)
        cb.start()
        ca.wait()
        cb.wait()
        for r in range(128):
            wout[pl.ds(r * 16, 16)] = jnp.exp(_leaky(abuf[r] + bbuf[r]))
        pltpu.sync_copy(wout.at[pl.ds(0, 2048)],
                        w_hbm.at[pl.ds((base + k * 128) * 16, 2048)])
        return carry

    lax.fori_loop(0, nchunk, chunk, 0)

    tail = EPW - nchunk * 128
    if tail:
        toff = nchunk * 128
        ca = pltpu.make_async_copy(
            asrc_hbm.at[src_t.at[pl.ds(toff, tail)]],
            abuf.at[pl.ds(0, tail)], sem_a)
        ca.start()
        cb = pltpu.make_async_copy(
            adst_hbm.at[dst_t.at[pl.ds(toff, tail)]],
            bbuf.at[pl.ds(0, tail)], sem_b)
        cb.start()
        ca.wait()
        cb.wait()
        for r in range(tail):
            wout[pl.ds(r * 16, 16)] = jnp.exp(_leaky(abuf[r] + bbuf[r]))
        pltpu.sync_copy(wout.at[pl.ds(0, tail * 16)],
                        w_hbm.at[pl.ds((base + toff) * 16, tail * 16)])


def _a1(asrc16, adst16, src, dst):
    mesh = plsc.VectorSubcoreMesh(core_axis_name="c", subcore_axis_name="s")
    f = pl.kernel(
        _a1_body,
        out_type=jax.ShapeDtypeStruct((E * 16,), jnp.float32),
        mesh=mesh,
        scratch_types=[
            pltpu.VMEM((EPW,), jnp.int32),
            pltpu.VMEM((EPW,), jnp.int32),
            pltpu.VMEM((128, 16), jnp.float32),
            pltpu.VMEM((128, 16), jnp.float32),
            pltpu.VMEM((2048,), jnp.float32),
            pltpu.SemaphoreType.DMA,
            pltpu.SemaphoreType.DMA,
        ],
        compiler_params=_SC_PARAMS,
    )
    return f(asrc16, adst16, src, dst)


def _b1_body(xp_hbm, w_hbm, src_hbm, dst_hbm, acc_hbm, den_hbm,
             srcb, dstb, loc_l, src_l, eid_l,
             xbA, xbB, wbA, wbB, zbuf, acc_sh, den_sh,
             semGA, semGB, semSA, semSB):
    dv = D1 // 16
    zrows = R1_PAD // NS
    frows = R1 // NS
    c = lax.axis_index("c")
    s = lax.axis_index("s")

    def zzb(i, carry):
        zbuf[i, pl.ds(0, 16)] = jnp.zeros((16,), jnp.float32)
        return carry
    lax.fori_loop(0, zrows, zzb, 0)

    def gstart(xb, wb, srcv, eidv, sem):
        pltpu.async_copy(xp_hbm.at[srcv], xb, sem)
        pltpu.async_copy(w_hbm.at[eidv], wb, sem)

    def gwait(xb, wb, sem):
        pltpu.make_async_copy(xp_hbm.at[pl.ds(0, CH)], xb, sem).wait()
        pltpu.make_async_copy(w_hbm.at[pl.ds(0, CH)], wb, sem).wait()

    def compute(xb, wb):
        for e in range(CH):
            wv = wb[e]
            for h in range(HEADS):
                ws = _splat(wv, h)
                for v in range(HIDDEN // 16):
                    col = h * HIDDEN + v * 16
                    xb[e, pl.ds(col, 16)] = xb[e, pl.ds(col, 16)] * ws

    def sstart(xb, wb, locv, sem):
        pltpu.async_copy(xb, acc_sh.at[locv], sem, add=True)
        pltpu.async_copy(wb, den_sh.at[locv], sem, add=True)

    def sdrain(xb, wb, sem):
        pltpu.make_async_copy(xb, acc_sh.at[pl.ds(0, CH)], sem).wait()
        pltpu.make_async_copy(wb, den_sh.at[pl.ds(0, CH)], sem).wait()

    def lists(i):
        return (src_l[pl.ds(i * CH, CH)], eid_l[pl.ds(i * CH, CH)],
                loc_l[pl.ds(i * CH, CH)])

    def one_pass(p, pcarry):
        lo = (p * NC + c) * R1

        def zx(v, carry):
            zero16 = jnp.zeros((16,), jnp.float32)
            for r in range(CH):
                xbA[r, pl.ds(v * 16, 16)] = zero16
            return carry
        lax.fori_loop(0, dv, zx, 0)
        nzc = zrows // CH
        ztail = zrows - nzc * CH
        for z in range(nzc):
            pltpu.sync_copy(xbA, acc_sh.at[pl.ds(s * zrows + z * CH, CH)])
        if ztail:
            pltpu.sync_copy(xbA.at[pl.ds(0, ztail)],
                            acc_sh.at[pl.ds(s * zrows + nzc * CH, ztail)])
        pltpu.sync_copy(zbuf, den_sh.at[pl.ds(s * zrows, zrows)])
        plsc.subcore_barrier()

        def one_block(q, qcarry):
            base = s * EPT + q * SB
            pltpu.sync_copy(src_hbm.at[pl.ds(base, SB)], srcb)
            pltpu.sync_copy(dst_hbm.at[pl.ds(base, SB)], dstb)

            def scan(i, cnt):
                d = dstb[pl.ds(i * 16, 16)]
                m = (d >= lo) & (d < lo + R1)
                plsc.store_compressed(loc_l.at[pl.ds(cnt, 16)], d - lo,
                                      mask=m)
                plsc.store_compressed(src_l.at[pl.ds(cnt, 16)],
                                      srcb[pl.ds(i * 16, 16)], mask=m)
                plsc.store_compressed(eid_l.at[pl.ds(cnt, 16)],
                                      base + i * 16 + _i16(), mask=m)
                return cnt + jnp.sum(m.astype(jnp.int32))
            cnt = lax.fori_loop(0, SB // 16, scan, 0)

            for t in range(2):
                loc_l[pl.ds(cnt + t * 16, 16)] = jnp.full(
                    (16,), R1, jnp.int32)
                src_l[pl.ds(cnt + t * 16, 16)] = jnp.zeros((16,), jnp.int32)
                eid_l[pl.ds(cnt + t * 16, 16)] = jnp.zeros((16,), jnp.int32)
            npair = (cnt + 2 * CH - 1) // (2 * CH)

            @pl.when(npair > 0)
            def _():
                sa, ea, la = lists(0)
                gstart(xbA, wbA, sa, ea, semGA)
                sb_, eb, lb = lists(1)
                gstart(xbB, wbB, sb_, eb, semGB)

            def pair(j, carry):
                sa, ea, la = lists(2 * j)
                gwait(xbA, wbA, semGA)
                compute(xbA, wbA)
                sstart(xbA, wbA, la, semSA)
                sb_, eb, lb = lists(2 * j + 1)
                gwait(xbB, wbB, semGB)
                compute(xbB, wbB)
                sstart(xbB, wbB, lb, semSB)

                @pl.when(j < npair - 1)
                def _():
                    sn, en, ln = lists(2 * j + 2)
                    sdrain(xbA, wbA, semSA)
                    gstart(xbA, wbA, sn, en, semGA)
                    sn2, en2, ln2 = lists(2 * j + 3)
                    sdrain(xbB, wbB, semSB)
                    gstart(xbB, wbB, sn2, en2, semGB)
                return carry
            lax.fori_loop(0, npair, pair, 0)

            @pl.when(npair > 0)
            def _():
                sdrain(xbA, wbA, semSA)
                sdrain(xbB, wbB, semSB)
            return qcarry
        lax.fori_loop(0, EPT // SB, one_block, 0)
        plsc.subcore_barrier()

        pltpu.sync_copy(acc_sh.at[pl.ds(s * frows, frows)],
                        acc_hbm.at[pl.ds(lo + s * frows, frows)])
        pltpu.sync_copy(den_sh.at[pl.ds(s * frows, frows)],
                        den_hbm.at[pl.ds(lo + s * frows, frows)])
        plsc.subcore_barrier()
        return pcarry
    lax.fori_loop(0, P1, one_pass, 0)


def _run_b1(xp, w, src, dst):
    mesh = plsc.VectorSubcoreMesh(core_axis_name="c", subcore_axis_name="s")
    out_rows = 2 * P1 * R1
    f = pl.kernel(
        _b1_body,
        out_type=[
            jax.ShapeDtypeStruct((out_rows, D1), jnp.float32),
            jax.ShapeDtypeStruct((out_rows, 16), jnp.float32),
        ],
        mesh=mesh,
        scratch_types=[
            pltpu.VMEM((SB,), jnp.int32),
            pltpu.VMEM((SB,), jnp.int32),
            pltpu.VMEM((SB + 64,), jnp.int32),
            pltpu.VMEM((SB + 64,), jnp.int32),
            pltpu.VMEM((SB + 64,), jnp.int32),
            pltpu.VMEM((CH, D1), jnp.float32),
            pltpu.VMEM((CH, D1), jnp.float32),
            pltpu.VMEM((CH, 16), jnp.float32),
            pltpu.VMEM((CH, 16), jnp.float32),
            pltpu.VMEM((R1_PAD // NS, 16), jnp.float32),
            pltpu.VMEM_SHARED((R1_PAD, D1), jnp.float32),
            pltpu.VMEM_SHARED((R1_PAD, 16), jnp.float32),
            pltpu.SemaphoreType.DMA,
            pltpu.SemaphoreType.DMA,
            pltpu.SemaphoreType.DMA,
            pltpu.SemaphoreType.DMA,
        ],
        compiler_params=_SC_PARAMS,
    )
    return f(xp, w, src, dst)


def _b2_body(xp_hbm, asrc_hbm, adst_hbm, src_hbm, dst_hbm, acc_hbm, den_hbm,
             srcb, dstb, loc_l, src_l,
             xbA, xbB, wbA, wbB, zbuf, asrc_t, adst_t, acc_sh, den_sh,
             semGA, semGB, semSA, semSB):
    dv = D2 // 16
    zrows = R2_PAD // NS
    frows = R2 // NS
    c = lax.axis_index("c")
    s = lax.axis_index("s")
    lo = c * R2 + N_CON
    pltpu.sync_copy(asrc_hbm, asrc_t)
    pltpu.sync_copy(adst_hbm, adst_t)

    def zzb(i, carry):
        zbuf[i, pl.ds(0, 16)] = jnp.zeros((16,), jnp.float32)
        return carry
    lax.fori_loop(0, zrows, zzb, 0)

    def zx(v, carry):
        zero16 = jnp.zeros((16,), jnp.float32)
        for r in range(CH):
            xbA[r, pl.ds(v * 16, 16)] = zero16
        return carry
    lax.fori_loop(0, dv, zx, 0)
    nzc = zrows // CH
    ztail = zrows - nzc * CH
    for z in range(nzc):
        pltpu.sync_copy(xbA, acc_sh.at[pl.ds(s * zrows + z * CH, CH)])
    if ztail:
        pltpu.sync_copy(xbA.at[pl.ds(0, ztail)],
                        acc_sh.at[pl.ds(s * zrows + nzc * CH, ztail)])
    pltpu.sync_copy(zbuf, den_sh.at[pl.ds(s * zrows, zrows)])
    plsc.subcore_barrier()

    def gstart(xb, wb, srcv, locv, sem):
        pltpu.async_copy(xp_hbm.at[srcv], xb, sem)
        av = plsc.load_gather(asrc_t, [srcv])
        bv = plsc.load_gather(adst_t, [jnp.minimum(locv + lo, N - 1)])
        w16 = jnp.exp(_leaky(av + bv))

        for e in range(CH):
            wb[e, pl.ds(0, 16)] = _splat(w16, e)

    def gwait(xb, sem):
        pltpu.make_async_copy(xp_hbm.at[pl.ds(0, CH)], xb, sem).wait()

    def compute(xb, wb):
        for e in range(CH):
            ws = wb[e]
            for v in range(dv):
                xb[e, pl.ds(v * 16, 16)] = xb[e, pl.ds(v * 16, 16)] * ws

    def sstart(xb, wb, locv, sem):
        pltpu.async_copy(xb, acc_sh.at[locv], sem, add=True)
        pltpu.async_copy(wb, den_sh.at[locv], sem, add=True)

    def sdrain(xb, wb, sem):
        pltpu.make_async_copy(xb, acc_sh.at[pl.ds(0, CH)], sem).wait()
        pltpu.make_async_copy(wb, den_sh.at[pl.ds(0, CH)], sem).wait()

    def lists(i):
        return (src_l[pl.ds(i * CH, CH)], loc_l[pl.ds(i * CH, CH)])

    def one_block(q, qcarry):
        base = s * EPT + q * SB
        pltpu.sync_copy(src_hbm.at[pl.ds(base, SB)], srcb)
        pltpu.sync_copy(dst_hbm.at[pl.ds(base, SB)], dstb)

        def scan(i, cnt):
            d = dstb[pl.ds(i * 16, 16)]
            m = (d >= lo) & (d < lo + R2)
            plsc.store_compressed(loc_l.at[pl.ds(cnt, 16)], d - lo, mask=m)
            plsc.store_compressed(src_l.at[pl.ds(cnt, 16)],
                                  srcb[pl.ds(i * 16, 16)], mask=m)
            return cnt + jnp.sum(m.astype(jnp.int32))
        cnt = lax.fori_loop(0, SB // 16, scan, 0)

        for t in range(2):
            loc_l[pl.ds(cnt + t * 16, 16)] = jnp.full((16,), R2, jnp.int32)
            src_l[pl.ds(cnt + t * 16, 16)] = jnp.zeros((16,), jnp.int32)
        npair = (cnt + 2 * CH - 1) // (2 * CH)

        @pl.when(npair > 0)
        def _():
            sa, la = lists(0)
            gstart(xbA, wbA, sa, la, semGA)
            sb_, lb = lists(1)
            gstart(xbB, wbB, sb_, lb, semGB)

        def pair(j, carry):
            sa, la = lists(2 * j)
            gwait(xbA, semGA)
            compute(xbA, wbA)
            sstart(xbA, wbA, la, semSA)
            sb_, lb = lists(2 * j + 1)
            gwait(xbB, semGB)
            compute(xbB, wbB)
            sstart(xbB, wbB, lb, semSB)

            @pl.when(j < npair - 1)
            def _():
                sn, ln = lists(2 * j + 2)
                sdrain(xbA, wbA, semSA)
                gstart(xbA, wbA, sn, ln, semGA)
                sn2, ln2 = lists(2 * j + 3)
                sdrain(xbB, wbB, semSB)
                gstart(xbB, wbB, sn2, ln2, semGB)
            return carry
        lax.fori_loop(0, npair, pair, 0)

        @pl.when(npair > 0)
        def _():
            sdrain(xbA, wbA, semSA)
            sdrain(xbB, wbB, semSB)
        return qcarry
    lax.fori_loop(0, EPT // SB, one_block, 0)
    plsc.subcore_barrier()

    pltpu.sync_copy(acc_sh.at[pl.ds(s * frows, frows)],
                    acc_hbm.at[pl.ds(c * R2 + s * frows, frows)])
    pltpu.sync_copy(den_sh.at[pl.ds(s * frows, frows)],
                    den_hbm.at[pl.ds(c * R2 + s * frows, frows)])
    plsc.subcore_barrier()


def _run_b2(xp2, asrc2, adst2, src, dst):
    mesh = plsc.VectorSubcoreMesh(core_axis_name="c", subcore_axis_name="s")
    f = pl.kernel(
        _b2_body,
        out_type=[
            jax.ShapeDtypeStruct((2 * R2, D2), jnp.float32),
            jax.ShapeDtypeStruct((2 * R2, 16), jnp.float32),
        ],
        mesh=mesh,
        scratch_types=[
            pltpu.VMEM((SB,), jnp.int32),
            pltpu.VMEM((SB,), jnp.int32),
            pltpu.VMEM((SB + 64,), jnp.int32),
            pltpu.VMEM((SB + 64,), jnp.int32),
            pltpu.VMEM((CH, D2), jnp.float32),
            pltpu.VMEM((CH, D2), jnp.float32),
            pltpu.VMEM((CH, 16), jnp.float32),
            pltpu.VMEM((CH, 16), jnp.float32),
            pltpu.VMEM((R2_PAD // NS, 16), jnp.float32),
            pltpu.VMEM((N,), jnp.float32),
            pltpu.VMEM((N,), jnp.float32),
            pltpu.VMEM_SHARED((R2_PAD, D2), jnp.float32),
            pltpu.VMEM_SHARED((R2_PAD, 16), jnp.float32),
            pltpu.SemaphoreType.DMA,
            pltpu.SemaphoreType.DMA,
            pltpu.SemaphoreType.DMA,
            pltpu.SemaphoreType.DMA,
        ],
        compiler_params=_SC_PARAMS,
    )
    return f(xp2, asrc2, adst2, src, dst)


def _tc2_body(acc_ref, den_ref, b1_ref, w2_ref, a2_ref, xp2_ref, att2_ref):
    feat = acc_ref[...].reshape(-1, HEADS, HIDDEN)
    den = den_ref[...][:, :HEADS].reshape(-1, HEADS, 1)
    emb = jax.nn.relu((feat / (den + 1e-16)).reshape(-1, D1) + b1_ref[...])
    xp2_ref[...] = jnp.dot(emb, w2_ref[...],
                           preferred_element_type=jnp.float32)
    att2_ref[...] = jnp.dot(emb, a2_ref[...],
                            preferred_element_type=jnp.float32)


def _tc2(acc1, den1, b1, W2, A2):
    blk = 1000
    return pl.pallas_call(
        _tc2_body,
        grid=(N // blk,),
        in_specs=[
            pl.BlockSpec((blk, D1), lambda i: (i, 0)),
            pl.BlockSpec((blk, 16), lambda i: (i, 0)),
            pl.BlockSpec((D1,), lambda i: (0,)),
            pl.BlockSpec((D1, HIDDEN), lambda i: (0, 0)),
            pl.BlockSpec((D1, 8), lambda i: (0, 0)),
        ],
        out_specs=[
            pl.BlockSpec((blk, HIDDEN), lambda i: (i, 0)),
            pl.BlockSpec((blk, 8), lambda i: (i, 0)),
        ],
        out_shape=[
            jax.ShapeDtypeStruct((N, HIDDEN), jnp.float32),
            jax.ShapeDtypeStruct((N, 8), jnp.float32),
        ],
    )(acc1, den1, b1, W2, A2)


def _tc3_body(acc_ref, den_ref, b2_ref, wo_ref, bo_ref, out_ref):
    den = den_ref[...][:, 0:1]
    emb = jax.nn.relu(acc_ref[...] / (den + 1e-16) + b2_ref[...])
    out_ref[...] = jnp.dot(emb, wo_ref[...],
                           preferred_element_type=jnp.float32) + bo_ref[...]


def _tc3(acc2, den2, b2, Wo, bo):
    blk = 1000
    return pl.pallas_call(
        _tc3_body,
        grid=(N_COL // blk,),
        in_specs=[
            pl.BlockSpec((blk, D2), lambda i: (i, 0)),
            pl.BlockSpec((blk, 16), lambda i: (i, 0)),
            pl.BlockSpec((HIDDEN,), lambda i: (0,)),
            pl.BlockSpec((HIDDEN, EMBED), lambda i: (0, 0)),
            pl.BlockSpec((EMBED,), lambda i: (0,)),
        ],
        out_specs=pl.BlockSpec((blk, EMBED), lambda i: (i, 0)),
        out_shape=jax.ShapeDtypeStruct((N_COL, EMBED), jnp.float32),
    )(acc2, den2, b2, Wo, bo)


def kernel(constraints_state, columns_state, edges, Wn, bn, Wc, bc, W1,
           a_src1, a_dst1, b1, W2, a_src2, a_dst2, b2, Wo, bo):
    Wnf = Wn[:HIDDEN] + Wn[HIDDEN:]
    Wcf = Wc[:HIDDEN] + Wc[HIDDEN:]
    As1 = jnp.einsum("khd,hd->kh", W1.reshape(HIDDEN, HEADS, HIDDEN), a_src1)
    Ad1 = jnp.einsum("khd,hd->kh", W1.reshape(HIDDEN, HEADS, HIDDEN), a_dst1)
    A32 = jnp.concatenate([As1, As1, Ad1, Ad1], axis=1)
    As2 = W2 @ a_src2[0]
    Ad2 = W2 @ a_dst2[0]
    A2 = jnp.stack([As2, Ad2] + [jnp.zeros_like(As2)] * 6, axis=1)

    src = edges[0]
    dst = edges[1]

    xp1, att = _tc1(constraints_state, columns_state, Wnf, bn, Wcf, bc,
                    W1, A32)
    w1flat = _a1(att[:, :16], att[:, 16:], src, dst)
    w1 = w1flat.reshape(E, 16)
    acc1, den1 = _run_b1(xp1, w1, src, dst)
    xp2, att2 = _tc2(acc1, den1, b1, W2, A2)
    acc2, den2 = _run_b2(xp2, att2[:, 0], att2[:, 1], src, dst)
    return _tc3(acc2, den2, b2, Wo, bo)
